# Initial kernel scaffold; baseline (speedup 1.0000x reference)
import jax
import jax.numpy as jnp
from jax.experimental import pallas as pl


def kernel(x, norm_w, Wr, W1s, b1s, W2s, b2s, Wgs, W1r, b1r, W2r, b2r, Wgr):
    raise NotImplementedError("write your pallas kernel here")


# speedup vs baseline: 2.5064x; 2.5064x over previous
import functools

import jax
import jax.numpy as jnp
from jax.experimental import pallas as pl
from jax.experimental.pallas import tpu as pltpu


def _dense_moe_body(n_routed, x_ref, nw_ref, wr_ref, w1_ref, wg_ref, w2_ref,
                    b1_ref, b2_ref, out_ref):
    x = x_ref[...]
    nw = nw_ref[...]
    ms = jnp.mean(x * x, axis=-1, keepdims=True)
    xn = x * nw * jax.lax.rsqrt(ms + 1e-6)
    aff = jnp.dot(xn, wr_ref[...], preferred_element_type=jnp.float32)
    iota = jax.lax.broadcasted_iota(jnp.int32, aff.shape, 1)
    m0 = jnp.max(aff, axis=-1, keepdims=True)
    i0 = jnp.min(jnp.where(aff == m0, iota, n_routed), axis=-1, keepdims=True)
    aff1 = jnp.where(iota == i0, -jnp.inf, aff)
    m1 = jnp.max(aff1, axis=-1, keepdims=True)
    i1 = jnp.min(jnp.where(aff1 == m1, iota, n_routed), axis=-1, keepdims=True)

    xnb = xn.astype(jnp.bfloat16)
    acc = x
    for e in range(n_routed + 1):
        h = jnp.dot(xnb, w1_ref[e], preferred_element_type=jnp.float32)
        h = h + b1_ref[e:e + 1]
        g = jnp.dot(h.astype(jnp.bfloat16), wg_ref[0 if e < n_routed else 1],
                    preferred_element_type=jnp.float32)
        h = h * (g * (1.0 / (1.0 + jnp.exp(-g))))
        y = jnp.dot(h.astype(jnp.bfloat16), w2_ref[e],
                    preferred_element_type=jnp.float32)
        y = y + b2_ref[e:e + 1]
        if e < n_routed:
            sc = jnp.where(i0 == e, m0, 0.0) + jnp.where(i1 == e, m1, 0.0)
            acc = acc + sc * y
        else:
            acc = acc + y
    out_ref[...] = acc


def kernel(x, norm_w, Wr, W1s, b1s, W2s, b2s, Wgs, W1r, b1r, W2r, b2r, Wgr):
    B, S, D = x.shape
    E = Wr.shape[-1]
    H = W1r.shape[-1]
    T = B * S
    BT = 512 if T % 512 == 0 else T

    xf = x.reshape(T, D)
    nw = norm_w.reshape(1, D)
    bf = jnp.bfloat16
    W1all = jnp.concatenate([W1r, W1s], axis=0).astype(bf)
    W2all = jnp.concatenate([W2r, W2s], axis=0).astype(bf)
    Wgall = jnp.stack([Wgr, Wgs], axis=0).astype(bf)
    b1all = jnp.concatenate([b1r, b1s], axis=0)[:, 0, :]
    b2all = jnp.concatenate([b2r, b2s], axis=0)[:, 0, :]

    grid = (T // BT,)
    out = pl.pallas_call(
        functools.partial(_dense_moe_body, E),
        grid=grid,
        in_specs=[
            pl.BlockSpec((BT, D), lambda i: (i, 0)),
            pl.BlockSpec((1, D), lambda i: (0, 0)),
            pl.BlockSpec((D, E), lambda i: (0, 0)),
            pl.BlockSpec((E + 1, D, H), lambda i: (0, 0, 0)),
            pl.BlockSpec((2, H, H), lambda i: (0, 0, 0)),
            pl.BlockSpec((E + 1, H, D), lambda i: (0, 0, 0)),
            pl.BlockSpec((E + 1, H), lambda i: (0, 0)),
            pl.BlockSpec((E + 1, D), lambda i: (0, 0)),
        ],
        out_specs=pl.BlockSpec((BT, D), lambda i: (i, 0)),
        out_shape=jax.ShapeDtypeStruct((T, D), jnp.float32),
    )(xf, nw, Wr, W1all, Wgall, W2all, b1all, b2all)
    return out.reshape(B, S, D)
